# baseline (speedup 1.0000x reference)
import functools

import jax
import jax.numpy as jnp
from jax import lax
from jax.experimental import pallas as pl
from jax.experimental.pallas import tpu as pltpu
from jax.experimental.pallas import tpu_sc as plsc

_N = 10000
_E = 320000
_H = 128
_NG = 200
_NC = 2
_NS = 16
_NW = _NC * _NS
_K = 80
_DEGW = 128
_EPW = _E // _NW
_NCHUNK = _N // _K
_TPS = -(-_NCHUNK // _NS)

_mesh = plsc.VectorSubcoreMesh(core_axis_name="c", subcore_axis_name="s")


@functools.partial(
    pl.kernel,
    out_type=(jax.ShapeDtypeStruct((_N, _H), jnp.float32),
              jax.ShapeDtypeStruct((_NC * _N, _DEGW), jnp.float32)),
    mesh=_mesh,
    scratch_types=[
        [pltpu.VMEM((_K,), jnp.int32) for _ in range(2)],
        [pltpu.VMEM((_K, _H), jnp.float32) for _ in range(2)],
        [pltpu.VMEM((_K,), jnp.int32) for _ in range(2)],
        pltpu.VMEM((_K, _DEGW), jnp.float32),
        pltpu.VMEM_SHARED((_N, _DEGW), jnp.float32),
        [pltpu.SemaphoreType.DMA for _ in range(2)],
        [pltpu.SemaphoreType.DMA for _ in range(2)],
    ],
)
def _sc_embed_deg(z_hbm, dst_hbm, table_hbm, zeros_hbm, ones_hbm,
                  x_hbm, deg_hbm, zi, rows, di, ones_v, dacc, esem, ssem):
    cid = lax.axis_index("c")
    sid = lax.axis_index("s")
    wid = sid * _NC + cid
    for t in range(_TPS):
        c = sid + t * _NS

        @pl.when(c < _NCHUNK)
        def _():
            pltpu.sync_copy(zeros_hbm, dacc.at[pl.ds(c * _K, _K)])

    pltpu.sync_copy(ones_hbm, ones_v)
    plsc.subcore_barrier()

    def chunk_id(t):
        c = wid + t * _NW
        return jnp.where(c < _NCHUNK, c, 0)

    def load_z(t, b):
        pltpu.sync_copy(z_hbm.at[pl.ds(chunk_id(t) * _K, _K)], zi[b])

    def write_x(t, b):
        pltpu.sync_copy(rows[b], x_hbm.at[pl.ds(chunk_id(t) * _K, _K)])

    load_z(0, 0)
    g0 = pltpu.async_copy(table_hbm.at[zi[0]], rows[0], esem[0])
    load_z(1, 1)
    g1 = pltpu.async_copy(table_hbm.at[zi[1]], rows[1], esem[1])
    g0.wait()
    write_x(0, 0)
    load_z(2, 0)
    g2 = pltpu.async_copy(table_hbm.at[zi[0]], rows[0], esem[0])
    g1.wait()
    write_x(1, 1)
    load_z(3, 1)
    g3 = pltpu.async_copy(table_hbm.at[zi[1]], rows[1], esem[1])
    g2.wait()
    write_x(2, 0)
    g3.wait()
    write_x(3, 1)

    ebase = cid * (_E // _NC) + sid * _EPW
    npair = (_EPW // _K) // 2
    nrem = _EPW // _K - 2 * npair

    def dpair(u, carry):
        off = ebase + 2 * u * _K
        pltpu.sync_copy(dst_hbm.at[pl.ds(off, _K)], di[0])
        s0 = pltpu.async_copy(ones_v, dacc.at[di[0]], ssem[0], add=True)
        pltpu.sync_copy(dst_hbm.at[pl.ds(off + _K, _K)], di[1])
        s1 = pltpu.async_copy(ones_v, dacc.at[di[1]], ssem[1], add=True)
        s0.wait()
        s1.wait()
        return carry

    lax.fori_loop(0, npair, dpair, 0)
    for r in range(nrem):
        off = ebase + (2 * npair + r) * _K
        pltpu.sync_copy(dst_hbm.at[pl.ds(off, _K)], di[0])
        pltpu.sync_copy(ones_v, dacc.at[di[0]], add=True)

    plsc.subcore_barrier()
    for t in range(_TPS):
        c = sid + t * _NS

        @pl.when(c < _NCHUNK)
        def _():
            pltpu.sync_copy(dacc.at[pl.ds(c * _K, _K)],
                            deg_hbm.at[pl.ds(cid * _N + c * _K, _K)])


_KA = 128
_CPW = _EPW // _KA
_TAIL = _EPW - _CPW * _KA
_G = 2
_NGRP = _CPW // _G


@functools.partial(
    pl.kernel,
    out_type=jax.ShapeDtypeStruct((_NC * _N, _H), jnp.float32),
    mesh=_mesh,
    scratch_types=[
        [pltpu.VMEM((_KA,), jnp.int32) for _ in range(_G)],
        [pltpu.VMEM((_KA,), jnp.int32) for _ in range(_G)],
        [pltpu.VMEM((_KA, _H), jnp.float32) for _ in range(_G)],
        pltpu.VMEM((_TAIL,), jnp.int32),
        pltpu.VMEM((_TAIL,), jnp.int32),
        pltpu.VMEM((_TAIL, _H), jnp.float32),
        pltpu.VMEM_SHARED((_N, _H), jnp.float32),
        [pltpu.SemaphoreType.DMA for _ in range(_G)],
        pltpu.SemaphoreType.DMA,
    ],
)
def _sc_aggregate(y_hbm, src_hbm, dst_hbm, zeros_hbm, p_hbm,
                  si, di, rows, si_t, di_t, rows_t, acc, gsem, ssem):
    cid = lax.axis_index("c")
    sid = lax.axis_index("s")
    for t in range(_TPS):
        c = sid + t * _NS

        @pl.when(c < _NCHUNK)
        def _():
            pltpu.sync_copy(zeros_hbm, acc.at[pl.ds(c * _K, _K)])

    plsc.subcore_barrier()
    ebase = cid * (_E // _NC) + sid * _EPW

    def group(gbase, nj):
        gds = []
        for j in range(nj):
            off = gbase + j * _KA
            pltpu.sync_copy(src_hbm.at[pl.ds(off, _KA)], si[j])
            pltpu.sync_copy(dst_hbm.at[pl.ds(off, _KA)], di[j])
            gds.append(pltpu.async_copy(y_hbm.at[si[j]], rows[j], gsem[j]))
        sds = []
        for j in range(nj):
            gds[j].wait()
            sds.append(pltpu.async_copy(rows[j], acc.at[di[j]], ssem,
                                        add=True))
        for j in range(nj):
            sds[j].wait()

    def body(g, carry):
        group(ebase + g * (_G * _KA), _G)
        return carry

    lax.fori_loop(0, _NGRP, body, 0)
    offt = ebase + _CPW * _KA
    pltpu.sync_copy(src_hbm.at[pl.ds(offt, _TAIL)], si_t)
    pltpu.sync_copy(dst_hbm.at[pl.ds(offt, _TAIL)], di_t)
    pltpu.async_copy(y_hbm.at[si_t], rows_t, gsem[0]).wait()
    pltpu.sync_copy(rows_t, acc.at[di_t], add=True)
    plsc.subcore_barrier()
    for t in range(_TPS):
        c = sid + t * _NS

        @pl.when(c < _NCHUNK)
        def _():
            pltpu.sync_copy(acc.at[pl.ds(c * _K, _K)],
                            p_hbm.at[pl.ds(cid * _N + c * _K, _K)])


_BM = 1000


def _tc_first_body(deg_ref, x_ref, w_ref, y_ref, dinv_ref):
    deg = deg_ref[0, :, :1] + deg_ref[1, :, :1] + 1.0
    dinv = lax.rsqrt(deg)
    dinv_ref[...] = dinv
    y_ref[...] = dinv * jnp.dot(x_ref[...], w_ref[...],
                                preferred_element_type=jnp.float32)


_tc_first = pl.pallas_call(
    _tc_first_body,
    grid=(_N // _BM,),
    in_specs=[
        pl.BlockSpec((2, _BM, _DEGW), lambda i: (0, i, 0)),
        pl.BlockSpec((_BM, _H), lambda i: (i, 0)),
        pl.BlockSpec((_H, _H), lambda i: (0, 0)),
    ],
    out_specs=[
        pl.BlockSpec((_BM, _H), lambda i: (i, 0)),
        pl.BlockSpec((_BM, 1), lambda i: (i, 0)),
    ],
    out_shape=[
        jax.ShapeDtypeStruct((_N, _H), jnp.float32),
        jax.ShapeDtypeStruct((_N, 1), jnp.float32),
    ],
)


def _tc_mid_body(p_ref, y_ref, dinv_ref, b_ref, w_ref, yo_ref):
    dinv = dinv_ref[...]
    x = jnp.maximum(dinv * (p_ref[0] + p_ref[1] + y_ref[...]) + b_ref[...],
                    0.0)
    yo_ref[...] = dinv * jnp.dot(x, w_ref[...],
                                 preferred_element_type=jnp.float32)


_tc_mid = pl.pallas_call(
    _tc_mid_body,
    grid=(_N // _BM,),
    in_specs=[
        pl.BlockSpec((2, _BM, _H), lambda i: (0, i, 0)),
        pl.BlockSpec((_BM, _H), lambda i: (i, 0)),
        pl.BlockSpec((_BM, 1), lambda i: (i, 0)),
        pl.BlockSpec((1, _H), lambda i: (0, 0)),
        pl.BlockSpec((_H, _H), lambda i: (0, 0)),
    ],
    out_specs=pl.BlockSpec((_BM, _H), lambda i: (i, 0)),
    out_shape=jax.ShapeDtypeStruct((_N, _H), jnp.float32),
)


def _tc_tail_body(ps0_ref, pd0_ref, ps1_ref, pd1_ref, ys_ref, yd_ref,
                  ds_ref, dd_ref, b_ref, w1_ref, b1_ref, w2_ref, b2_ref,
                  o_ref):
    xs = ds_ref[...] * (ps0_ref[...] + ps1_ref[...] + ys_ref[...]) + b_ref[...]
    xd = dd_ref[...] * (pd0_ref[...] + pd1_ref[...] + yd_ref[...]) + b_ref[...]
    prod = xs * xd
    h = jnp.maximum(jnp.dot(prod, w1_ref[...],
                            preferred_element_type=jnp.float32) + b1_ref[...],
                    0.0)
    o_ref[...] = (jnp.dot(h, w2_ref[...], preferred_element_type=jnp.float32)
                  + b2_ref[...])


_tc_tail = pl.pallas_call(
    _tc_tail_body,
    out_shape=jax.ShapeDtypeStruct((_NG, 1), jnp.float32),
)


def kernel(z, edge_index, batch, z_table, W0, b0, W1, b1, W2, b2,
           lin1_W, lin1_b, lin2_W, lin2_b):
    z = z.astype(jnp.int32)
    src = edge_index[0].astype(jnp.int32)
    dst = edge_index[1].astype(jnp.int32)
    zeros_h = jnp.zeros((_K, _H), jnp.float32)
    ones16 = jnp.ones((_K, _DEGW), jnp.float32)

    x0, deg2 = _sc_embed_deg(z, dst, z_table, zeros_h, ones16)
    deg2 = deg2.reshape(_NC, _N, _DEGW)
    y, dinv = _tc_first(deg2, x0, W0)
    p = _sc_aggregate(y, src, dst, zeros_h).reshape(_NC, _N, _H)
    y = _tc_mid(p, y, dinv, b0.reshape(1, _H), W1)
    p = _sc_aggregate(y, src, dst, zeros_h).reshape(_NC, _N, _H)
    y = _tc_mid(p, y, dinv, b1.reshape(1, _H), W2)
    p = _sc_aggregate(y, src, dst, zeros_h).reshape(_NC, _N, _H)
    gs = _N // _NG
    p4 = p.reshape(_NC, _NG, gs, _H)
    y4 = y.reshape(_NG, gs, _H)
    d4 = dinv.reshape(_NG, gs, 1)
    out = _tc_tail(p4[0, :, 0], p4[0, :, 1], p4[1, :, 0], p4[1, :, 1],
                   y4[:, 0], y4[:, 1], d4[:, 0], d4[:, 1],
                   b2.reshape(1, _H), lin1_W, lin1_b.reshape(1, _H),
                   lin2_W, lin2_b.reshape(1, 1))
    return out


# speedup vs baseline: 1.0055x; 1.0055x over previous
import functools

import jax
import jax.numpy as jnp
from jax import lax
from jax.experimental import pallas as pl
from jax.experimental.pallas import tpu as pltpu
from jax.experimental.pallas import tpu_sc as plsc

_N = 10000
_E = 320000
_H = 128
_NG = 200
_NC = 2
_NS = 16
_NW = _NC * _NS
_K = 80
_DEGW = 128
_EPW = _E // _NW
_NCHUNK = _N // _K
_TPS = -(-_NCHUNK // _NS)

_mesh = plsc.VectorSubcoreMesh(core_axis_name="c", subcore_axis_name="s")


@functools.partial(
    pl.kernel,
    out_type=(jax.ShapeDtypeStruct((_N, _H), jnp.float32),
              jax.ShapeDtypeStruct((_NC * _N, _DEGW), jnp.float32)),
    mesh=_mesh,
    scratch_types=[
        [pltpu.VMEM((_K,), jnp.int32) for _ in range(2)],
        [pltpu.VMEM((_K, _H), jnp.float32) for _ in range(2)],
        [pltpu.VMEM((_K,), jnp.int32) for _ in range(2)],
        pltpu.VMEM((_K, _DEGW), jnp.float32),
        pltpu.VMEM_SHARED((_N, _DEGW), jnp.float32),
        [pltpu.SemaphoreType.DMA for _ in range(2)],
        [pltpu.SemaphoreType.DMA for _ in range(2)],
    ],
)
def _sc_embed_deg(z_hbm, dst_hbm, table_hbm, zeros_hbm, ones_hbm,
                  x_hbm, deg_hbm, zi, rows, di, ones_v, dacc, esem, ssem):
    cid = lax.axis_index("c")
    sid = lax.axis_index("s")
    wid = sid * _NC + cid
    for t in range(_TPS):
        c = sid + t * _NS

        @pl.when(c < _NCHUNK)
        def _():
            pltpu.sync_copy(zeros_hbm, dacc.at[pl.ds(c * _K, _K)])

    pltpu.sync_copy(ones_hbm, ones_v)
    plsc.subcore_barrier()

    def chunk_id(t):
        c = wid + t * _NW
        return jnp.where(c < _NCHUNK, c, 0)

    def load_z(t, b):
        pltpu.sync_copy(z_hbm.at[pl.ds(chunk_id(t) * _K, _K)], zi[b])

    def write_x(t, b):
        pltpu.sync_copy(rows[b], x_hbm.at[pl.ds(chunk_id(t) * _K, _K)])

    load_z(0, 0)
    g0 = pltpu.async_copy(table_hbm.at[zi[0]], rows[0], esem[0])
    load_z(1, 1)
    g1 = pltpu.async_copy(table_hbm.at[zi[1]], rows[1], esem[1])
    g0.wait()
    write_x(0, 0)
    load_z(2, 0)
    g2 = pltpu.async_copy(table_hbm.at[zi[0]], rows[0], esem[0])
    g1.wait()
    write_x(1, 1)
    load_z(3, 1)
    g3 = pltpu.async_copy(table_hbm.at[zi[1]], rows[1], esem[1])
    g2.wait()
    write_x(2, 0)
    g3.wait()
    write_x(3, 1)

    ebase = cid * (_E // _NC) + sid * _EPW
    npair = (_EPW // _K) // 2
    nrem = _EPW // _K - 2 * npair

    def dpair(u, carry):
        off = ebase + 2 * u * _K
        pltpu.sync_copy(dst_hbm.at[pl.ds(off, _K)], di[0])
        s0 = pltpu.async_copy(ones_v, dacc.at[di[0]], ssem[0], add=True)
        pltpu.sync_copy(dst_hbm.at[pl.ds(off + _K, _K)], di[1])
        s1 = pltpu.async_copy(ones_v, dacc.at[di[1]], ssem[1], add=True)
        s0.wait()
        s1.wait()
        return carry

    lax.fori_loop(0, npair, dpair, 0)
    for r in range(nrem):
        off = ebase + (2 * npair + r) * _K
        pltpu.sync_copy(dst_hbm.at[pl.ds(off, _K)], di[0])
        pltpu.sync_copy(ones_v, dacc.at[di[0]], add=True)

    plsc.subcore_barrier()
    for t in range(_TPS):
        c = sid + t * _NS

        @pl.when(c < _NCHUNK)
        def _():
            pltpu.sync_copy(dacc.at[pl.ds(c * _K, _K)],
                            deg_hbm.at[pl.ds(cid * _N + c * _K, _K)])


_KA = 128
_CPW = _EPW // _KA
_TAIL = _EPW - _CPW * _KA
_G = 2
_NGRP = _CPW // _G


@functools.partial(
    pl.kernel,
    out_type=jax.ShapeDtypeStruct((_NC * _N, _H), jnp.float32),
    mesh=_mesh,
    scratch_types=[
        [pltpu.VMEM((_KA,), jnp.int32) for _ in range(_G)],
        [pltpu.VMEM((_KA,), jnp.int32) for _ in range(_G)],
        [pltpu.VMEM((_KA, _H), jnp.float32) for _ in range(_G)],
        pltpu.VMEM((_TAIL,), jnp.int32),
        pltpu.VMEM((_TAIL,), jnp.int32),
        pltpu.VMEM((_TAIL, _H), jnp.float32),
        pltpu.VMEM_SHARED((_N, _H), jnp.float32),
        [pltpu.SemaphoreType.DMA for _ in range(_G)],
        pltpu.SemaphoreType.DMA,
    ],
)
def _sc_aggregate(y_hbm, src_hbm, dst_hbm, zeros_hbm, p_hbm,
                  si, di, rows, si_t, di_t, rows_t, acc, gsem, ssem):
    cid = lax.axis_index("c")
    sid = lax.axis_index("s")
    for t in range(_TPS):
        c = sid + t * _NS

        @pl.when(c < _NCHUNK)
        def _():
            pltpu.sync_copy(zeros_hbm, acc.at[pl.ds(c * _K, _K)])

    plsc.subcore_barrier()
    ebase = cid * (_E // _NC) + sid * _EPW

    def group(gbase, nj):
        gds = []
        for j in range(nj):
            off = gbase + j * _KA
            pltpu.sync_copy(src_hbm.at[pl.ds(off, _KA)], si[j])
            pltpu.sync_copy(dst_hbm.at[pl.ds(off, _KA)], di[j])
            gds.append(pltpu.async_copy(y_hbm.at[si[j]], rows[j], gsem[j]))
        sds = []
        for j in range(nj):
            gds[j].wait()
            sds.append(pltpu.async_copy(rows[j], acc.at[di[j]], ssem,
                                        add=True))
        for j in range(nj):
            sds[j].wait()

    def body(g, carry):
        group(ebase + g * (_G * _KA), _G)
        return carry

    lax.fori_loop(0, _NGRP, body, 0)
    offt = ebase + _CPW * _KA
    pltpu.sync_copy(src_hbm.at[pl.ds(offt, _TAIL)], si_t)
    pltpu.sync_copy(dst_hbm.at[pl.ds(offt, _TAIL)], di_t)
    pltpu.async_copy(y_hbm.at[si_t], rows_t, gsem[0]).wait()
    pltpu.sync_copy(rows_t, acc.at[di_t], add=True)
    plsc.subcore_barrier()
    for t in range(_TPS):
        c = sid + t * _NS

        @pl.when(c < _NCHUNK)
        def _():
            pltpu.sync_copy(acc.at[pl.ds(c * _K, _K)],
                            p_hbm.at[pl.ds(cid * _N + c * _K, _K)])


@functools.partial(
    pl.kernel,
    out_type=jax.ShapeDtypeStruct((2 * _NG, _H), jnp.float32),
    mesh=_mesh,
    scratch_types=[
        pltpu.VMEM((16,), jnp.int32),
        pltpu.VMEM((16, _H), jnp.float32),
        pltpu.SemaphoreType.DMA,
    ],
)
def _sc_pool(x_hbm, ci_hbm, g_hbm, idx_v, rows_v, sem):
    cid = lax.axis_index("c")
    sid = lax.axis_index("s")
    wid = sid * _NC + cid

    @pl.when(wid < (2 * _NG) // 16)
    def _():
        pltpu.sync_copy(ci_hbm.at[pl.ds(wid * 16, 16)], idx_v)
        pltpu.async_copy(x_hbm.at[idx_v], rows_v, sem).wait()
        pltpu.sync_copy(rows_v, g_hbm.at[pl.ds(wid * 16, 16)])


_BM = 1000


def _tc_first_body(deg_ref, x_ref, w_ref, y_ref, dinv_ref):
    deg = deg_ref[0, :, :1] + deg_ref[1, :, :1] + 1.0
    dinv = lax.rsqrt(deg)
    dinv_ref[...] = dinv
    y_ref[...] = dinv * jnp.dot(x_ref[...], w_ref[...],
                                preferred_element_type=jnp.float32)


_tc_first = pl.pallas_call(
    _tc_first_body,
    grid=(_N // _BM,),
    in_specs=[
        pl.BlockSpec((2, _BM, _DEGW), lambda i: (0, i, 0)),
        pl.BlockSpec((_BM, _H), lambda i: (i, 0)),
        pl.BlockSpec((_H, _H), lambda i: (0, 0)),
    ],
    out_specs=[
        pl.BlockSpec((_BM, _H), lambda i: (i, 0)),
        pl.BlockSpec((_BM, 1), lambda i: (i, 0)),
    ],
    out_shape=[
        jax.ShapeDtypeStruct((_N, _H), jnp.float32),
        jax.ShapeDtypeStruct((_N, 1), jnp.float32),
    ],
)


def _tc_mid_body(p_ref, y_ref, dinv_ref, b_ref, w_ref, yo_ref):
    dinv = dinv_ref[...]
    x = jnp.maximum(dinv * (p_ref[0] + p_ref[1] + y_ref[...]) + b_ref[...],
                    0.0)
    yo_ref[...] = dinv * jnp.dot(x, w_ref[...],
                                 preferred_element_type=jnp.float32)


_tc_mid = pl.pallas_call(
    _tc_mid_body,
    grid=(_N // _BM,),
    in_specs=[
        pl.BlockSpec((2, _BM, _H), lambda i: (0, i, 0)),
        pl.BlockSpec((_BM, _H), lambda i: (i, 0)),
        pl.BlockSpec((_BM, 1), lambda i: (i, 0)),
        pl.BlockSpec((1, _H), lambda i: (0, 0)),
        pl.BlockSpec((_H, _H), lambda i: (0, 0)),
    ],
    out_specs=pl.BlockSpec((_BM, _H), lambda i: (i, 0)),
    out_shape=jax.ShapeDtypeStruct((_N, _H), jnp.float32),
)


def _tc_final_body(p_ref, y_ref, dinv_ref, b_ref, x3_ref):
    x3_ref[...] = (dinv_ref[...] * (p_ref[0] + p_ref[1] + y_ref[...])
                   + b_ref[...])


_tc_final = pl.pallas_call(
    _tc_final_body,
    grid=(_N // _BM,),
    in_specs=[
        pl.BlockSpec((2, _BM, _H), lambda i: (0, i, 0)),
        pl.BlockSpec((_BM, _H), lambda i: (i, 0)),
        pl.BlockSpec((_BM, 1), lambda i: (i, 0)),
        pl.BlockSpec((1, _H), lambda i: (0, 0)),
    ],
    out_specs=pl.BlockSpec((_BM, _H), lambda i: (i, 0)),
    out_shape=jax.ShapeDtypeStruct((_N, _H), jnp.float32),
)


def _tc_mlp_body(g_ref, w1_ref, b1_ref, w2_ref, b2_ref, o_ref):
    prod = g_ref[:_NG] * g_ref[_NG:]
    h = jnp.maximum(jnp.dot(prod, w1_ref[...],
                            preferred_element_type=jnp.float32) + b1_ref[...],
                    0.0)
    o_ref[...] = (jnp.dot(h, w2_ref[...], preferred_element_type=jnp.float32)
                  + b2_ref[...])


_tc_mlp = pl.pallas_call(
    _tc_mlp_body,
    out_shape=jax.ShapeDtypeStruct((_NG, 1), jnp.float32),
)


def kernel(z, edge_index, batch, z_table, W0, b0, W1, b1, W2, b2,
           lin1_W, lin1_b, lin2_W, lin2_b):
    z = z.astype(jnp.int32)
    src = edge_index[0].astype(jnp.int32)
    dst = edge_index[1].astype(jnp.int32)
    ci = jnp.searchsorted(batch, jnp.arange(_NG, dtype=batch.dtype))
    poolidx = jnp.concatenate([ci, ci + 1]).astype(jnp.int32)
    zeros_h = jnp.zeros((_K, _H), jnp.float32)
    ones16 = jnp.ones((_K, _DEGW), jnp.float32)

    x0, deg2 = _sc_embed_deg(z, dst, z_table, zeros_h, ones16)
    deg2 = deg2.reshape(_NC, _N, _DEGW)
    y, dinv = _tc_first(deg2, x0, W0)
    p = _sc_aggregate(y, src, dst, zeros_h).reshape(_NC, _N, _H)
    y = _tc_mid(p, y, dinv, b0.reshape(1, _H), W1)
    p = _sc_aggregate(y, src, dst, zeros_h).reshape(_NC, _N, _H)
    y = _tc_mid(p, y, dinv, b1.reshape(1, _H), W2)
    p = _sc_aggregate(y, src, dst, zeros_h).reshape(_NC, _N, _H)
    x3 = _tc_final(p, y, dinv, b2.reshape(1, _H))
    g = _sc_pool(x3, poolidx)
    out = _tc_mlp(g, lin1_W, lin1_b.reshape(1, _H), lin2_W,
                  lin2_b.reshape(1, 1))
    return out


# speedup vs baseline: 1.0196x; 1.0140x over previous
import functools

import jax
import jax.numpy as jnp
from jax import lax
from jax.experimental import pallas as pl
from jax.experimental.pallas import tpu as pltpu
from jax.experimental.pallas import tpu_sc as plsc

_N = 10000
_E = 320000
_H = 128
_NG = 200
_NC = 2
_NS = 16
_NW = _NC * _NS
_K = 80
_DEGW = 128
_EPW = _E // _NW
_NCHUNK = _N // _K
_TPS = -(-_NCHUNK // _NS)

_mesh = plsc.VectorSubcoreMesh(core_axis_name="c", subcore_axis_name="s")


@functools.partial(
    pl.kernel,
    out_type=(jax.ShapeDtypeStruct((_N, _H), jnp.float32),
              jax.ShapeDtypeStruct((_NC * _N, _DEGW), jnp.float32)),
    mesh=_mesh,
    scratch_types=[
        [pltpu.VMEM((_K,), jnp.int32) for _ in range(2)],
        [pltpu.VMEM((_K, _H), jnp.float32) for _ in range(2)],
        [pltpu.VMEM((_K,), jnp.int32) for _ in range(2)],
        pltpu.VMEM((_K, _DEGW), jnp.float32),
        pltpu.VMEM_SHARED((_N, _DEGW), jnp.float32),
        [pltpu.SemaphoreType.DMA for _ in range(2)],
        [pltpu.SemaphoreType.DMA for _ in range(2)],
    ],
)
def _sc_embed_deg(z_hbm, dst_hbm, table_hbm, zeros_hbm, ones_hbm,
                  x_hbm, deg_hbm, zi, rows, di, ones_v, dacc, esem, ssem):
    cid = lax.axis_index("c")
    sid = lax.axis_index("s")
    wid = sid * _NC + cid
    for t in range(_TPS):
        c = sid + t * _NS

        @pl.when(c < _NCHUNK)
        def _():
            pltpu.sync_copy(zeros_hbm, dacc.at[pl.ds(c * _K, _K)])

    pltpu.sync_copy(ones_hbm, ones_v)
    plsc.subcore_barrier()

    def chunk_id(t):
        c = wid + t * _NW
        return jnp.where(c < _NCHUNK, c, 0)

    def load_z(t, b):
        pltpu.sync_copy(z_hbm.at[pl.ds(chunk_id(t) * _K, _K)], zi[b])

    def write_x(t, b):
        pltpu.sync_copy(rows[b], x_hbm.at[pl.ds(chunk_id(t) * _K, _K)])

    load_z(0, 0)
    g0 = pltpu.async_copy(table_hbm.at[zi[0]], rows[0], esem[0])
    load_z(1, 1)
    g1 = pltpu.async_copy(table_hbm.at[zi[1]], rows[1], esem[1])
    g0.wait()
    write_x(0, 0)
    load_z(2, 0)
    g2 = pltpu.async_copy(table_hbm.at[zi[0]], rows[0], esem[0])
    g1.wait()
    write_x(1, 1)
    load_z(3, 1)
    g3 = pltpu.async_copy(table_hbm.at[zi[1]], rows[1], esem[1])
    g2.wait()
    write_x(2, 0)
    g3.wait()
    write_x(3, 1)

    ebase = cid * (_E // _NC) + sid * _EPW
    npair = (_EPW // _K) // 2
    nrem = _EPW // _K - 2 * npair

    def dpair(u, carry):
        off = ebase + 2 * u * _K
        pltpu.sync_copy(dst_hbm.at[pl.ds(off, _K)], di[0])
        s0 = pltpu.async_copy(ones_v, dacc.at[di[0]], ssem[0], add=True)
        pltpu.sync_copy(dst_hbm.at[pl.ds(off + _K, _K)], di[1])
        s1 = pltpu.async_copy(ones_v, dacc.at[di[1]], ssem[1], add=True)
        s0.wait()
        s1.wait()
        return carry

    lax.fori_loop(0, npair, dpair, 0)
    for r in range(nrem):
        off = ebase + (2 * npair + r) * _K
        pltpu.sync_copy(dst_hbm.at[pl.ds(off, _K)], di[0])
        pltpu.sync_copy(ones_v, dacc.at[di[0]], add=True)

    plsc.subcore_barrier()
    for t in range(_TPS):
        c = sid + t * _NS

        @pl.when(c < _NCHUNK)
        def _():
            pltpu.sync_copy(dacc.at[pl.ds(c * _K, _K)],
                            deg_hbm.at[pl.ds(cid * _N + c * _K, _K)])


_KA = 128
_CPW = _EPW // _KA
_TAIL = _EPW - _CPW * _KA
_G = 2
_NGRP = _CPW // _G


@functools.partial(
    pl.kernel,
    out_type=jax.ShapeDtypeStruct((_NC * _N, _H), jnp.float32),
    mesh=_mesh,
    scratch_types=[
        [pltpu.VMEM((_KA,), jnp.int32) for _ in range(_G)],
        [pltpu.VMEM((_KA,), jnp.int32) for _ in range(_G)],
        [pltpu.VMEM((_KA, _H), jnp.float32) for _ in range(_G)],
        pltpu.VMEM((_TAIL,), jnp.int32),
        pltpu.VMEM((_TAIL,), jnp.int32),
        pltpu.VMEM((_TAIL, _H), jnp.float32),
        pltpu.VMEM_SHARED((_N, _H), jnp.float32),
        [pltpu.SemaphoreType.DMA for _ in range(_G)],
        pltpu.SemaphoreType.DMA,
    ],
)
def _sc_aggregate(y_hbm, src_hbm, dst_hbm, zeros_hbm, p_hbm,
                  si, di, rows, si_t, di_t, rows_t, acc, gsem, ssem):
    cid = lax.axis_index("c")
    sid = lax.axis_index("s")
    for t in range(_TPS):
        c = sid + t * _NS

        @pl.when(c < _NCHUNK)
        def _():
            pltpu.sync_copy(zeros_hbm, acc.at[pl.ds(c * _K, _K)])

    plsc.subcore_barrier()
    ebase = cid * (_E // _NC) + sid * _EPW

    def group(gbase, nj):
        gds = []
        for j in range(nj):
            off = gbase + j * _KA
            pltpu.sync_copy(src_hbm.at[pl.ds(off, _KA)], si[j])
            pltpu.sync_copy(dst_hbm.at[pl.ds(off, _KA)], di[j])
            gds.append(pltpu.async_copy(y_hbm.at[si[j]], rows[j], gsem[j]))
        sds = []
        for j in range(nj):
            gds[j].wait()
            sds.append(pltpu.async_copy(rows[j], acc.at[di[j]], ssem,
                                        add=True))
        for j in range(nj):
            sds[j].wait()

    def body(g, carry):
        group(ebase + g * (_G * _KA), _G)
        return carry

    lax.fori_loop(0, _NGRP, body, 0)
    offt = ebase + _CPW * _KA
    pltpu.sync_copy(src_hbm.at[pl.ds(offt, _TAIL)], si_t)
    pltpu.sync_copy(dst_hbm.at[pl.ds(offt, _TAIL)], di_t)
    pltpu.async_copy(y_hbm.at[si_t], rows_t, gsem[0]).wait()
    pltpu.sync_copy(rows_t, acc.at[di_t], add=True)
    plsc.subcore_barrier()
    for t in range(_TPS):
        c = sid + t * _NS

        @pl.when(c < _NCHUNK)
        def _():
            pltpu.sync_copy(acc.at[pl.ds(c * _K, _K)],
                            p_hbm.at[pl.ds(cid * _N + c * _K, _K)])


@functools.partial(
    pl.kernel,
    out_type=(jax.ShapeDtypeStruct((2, 2 * _NG, _H), jnp.float32),
              jax.ShapeDtypeStruct((2 * _NG, _H), jnp.float32),
              jax.ShapeDtypeStruct((2, 2 * _NG, _DEGW), jnp.float32)),
    mesh=_mesh,
    scratch_types=[
        pltpu.VMEM((16,), jnp.int32),
        [pltpu.VMEM((16, _H), jnp.float32) for _ in range(5)],
        [pltpu.SemaphoreType.DMA for _ in range(5)],
    ],
)
def _sc_pool(p_hbm, y_hbm, deg_hbm, ci_hbm, gp_hbm, gy_hbm, gd_hbm,
             idx_v, rows, sem):
    cid = lax.axis_index("c")
    sid = lax.axis_index("s")
    wid = sid * _NC + cid

    @pl.when(wid < (2 * _NG) // 16)
    def _():
        pltpu.sync_copy(ci_hbm.at[pl.ds(wid * 16, 16)], idx_v)
        ds = []
        for h in range(2):
            ds.append(pltpu.async_copy(
                p_hbm.at[pl.ds(h * _N, _N)].at[idx_v], rows[h], sem[h]))
            ds.append(pltpu.async_copy(
                deg_hbm.at[pl.ds(h * _N, _N)].at[idx_v], rows[2 + h],
                sem[2 + h]))
        ds.append(pltpu.async_copy(y_hbm.at[idx_v], rows[4], sem[4]))
        for d in ds:
            d.wait()
        for h in range(2):
            pltpu.sync_copy(rows[h], gp_hbm.at[h, pl.ds(wid * 16, 16)])
            pltpu.sync_copy(rows[2 + h], gd_hbm.at[h, pl.ds(wid * 16, 16)])
        pltpu.sync_copy(rows[4], gy_hbm.at[pl.ds(wid * 16, 16)])


_BM = 1000


def _tc_first_body(deg_ref, x_ref, w_ref, y_ref, dinv_ref):
    deg = deg_ref[0, :, :1] + deg_ref[1, :, :1] + 1.0
    dinv = lax.rsqrt(deg)
    dinv_ref[...] = dinv
    y_ref[...] = dinv * jnp.dot(x_ref[...], w_ref[...],
                                preferred_element_type=jnp.float32)


_tc_first = pl.pallas_call(
    _tc_first_body,
    grid=(_N // _BM,),
    in_specs=[
        pl.BlockSpec((2, _BM, _DEGW), lambda i: (0, i, 0)),
        pl.BlockSpec((_BM, _H), lambda i: (i, 0)),
        pl.BlockSpec((_H, _H), lambda i: (0, 0)),
    ],
    out_specs=[
        pl.BlockSpec((_BM, _H), lambda i: (i, 0)),
        pl.BlockSpec((_BM, 1), lambda i: (i, 0)),
    ],
    out_shape=[
        jax.ShapeDtypeStruct((_N, _H), jnp.float32),
        jax.ShapeDtypeStruct((_N, 1), jnp.float32),
    ],
)


def _tc_mid_body(p_ref, y_ref, dinv_ref, b_ref, w_ref, yo_ref):
    dinv = dinv_ref[...]
    x = jnp.maximum(dinv * (p_ref[0] + p_ref[1] + y_ref[...]) + b_ref[...],
                    0.0)
    yo_ref[...] = dinv * jnp.dot(x, w_ref[...],
                                 preferred_element_type=jnp.float32)


_tc_mid = pl.pallas_call(
    _tc_mid_body,
    grid=(_N // _BM,),
    in_specs=[
        pl.BlockSpec((2, _BM, _H), lambda i: (0, i, 0)),
        pl.BlockSpec((_BM, _H), lambda i: (i, 0)),
        pl.BlockSpec((_BM, 1), lambda i: (i, 0)),
        pl.BlockSpec((1, _H), lambda i: (0, 0)),
        pl.BlockSpec((_H, _H), lambda i: (0, 0)),
    ],
    out_specs=pl.BlockSpec((_BM, _H), lambda i: (i, 0)),
    out_shape=jax.ShapeDtypeStruct((_N, _H), jnp.float32),
)


def _tc_tail_body(gp_ref, gy_ref, gd_ref, b_ref, w1_ref, b1_ref,
                  w2_ref, b2_ref, o_ref):
    deg = gd_ref[0, :, :1] + gd_ref[1, :, :1] + 1.0
    dinv = lax.rsqrt(deg)
    x3 = dinv * (gp_ref[0] + gp_ref[1] + gy_ref[...]) + b_ref[...]
    prod = x3[:_NG] * x3[_NG:]
    h = jnp.maximum(jnp.dot(prod, w1_ref[...],
                            preferred_element_type=jnp.float32) + b1_ref[...],
                    0.0)
    o_ref[...] = (jnp.dot(h, w2_ref[...], preferred_element_type=jnp.float32)
                  + b2_ref[...])


_tc_tail = pl.pallas_call(
    _tc_tail_body,
    out_shape=jax.ShapeDtypeStruct((_NG, 1), jnp.float32),
)


def kernel(z, edge_index, batch, z_table, W0, b0, W1, b1, W2, b2,
           lin1_W, lin1_b, lin2_W, lin2_b):
    z = z.astype(jnp.int32)
    src = edge_index[0].astype(jnp.int32)
    dst = edge_index[1].astype(jnp.int32)
    ci = jnp.searchsorted(batch, jnp.arange(_NG, dtype=batch.dtype))
    poolidx = jnp.concatenate([ci, ci + 1]).astype(jnp.int32)
    zeros_h = jnp.zeros((_K, _H), jnp.float32)
    ones16 = jnp.ones((_K, _DEGW), jnp.float32)

    x0, deg2 = _sc_embed_deg(z, dst, z_table, zeros_h, ones16)
    y, dinv = _tc_first(deg2.reshape(_NC, _N, _DEGW), x0, W0)
    p = _sc_aggregate(y, src, dst, zeros_h)
    y = _tc_mid(p.reshape(_NC, _N, _H), y, dinv, b0.reshape(1, _H), W1)
    p = _sc_aggregate(y, src, dst, zeros_h)
    y = _tc_mid(p.reshape(_NC, _N, _H), y, dinv, b1.reshape(1, _H), W2)
    p = _sc_aggregate(y, src, dst, zeros_h)
    gp, gy, gd = _sc_pool(p, y, deg2, poolidx)
    out = _tc_tail(gp, gy, gd, b2.reshape(1, _H), lin1_W,
                   lin1_b.reshape(1, _H), lin2_W, lin2_b.reshape(1, 1))
    return out


# speedup vs baseline: 1.0953x; 1.0743x over previous
import functools

import jax
import jax.numpy as jnp
from jax import lax
from jax.experimental import pallas as pl
from jax.experimental.pallas import tpu as pltpu
from jax.experimental.pallas import tpu_sc as plsc

_N = 10000
_E = 320000
_H = 128
_NG = 200
_NC = 2
_NS = 16
_NW = _NC * _NS
_K = 80
_DEGW = 128
_EPW = _E // _NW
_NCHUNK = _N // _K
_TPS = -(-_NCHUNK // _NS)

_mesh = plsc.VectorSubcoreMesh(core_axis_name="c", subcore_axis_name="s")


@functools.partial(
    pl.kernel,
    out_type=(jax.ShapeDtypeStruct((_N, _H), jnp.float32),
              jax.ShapeDtypeStruct((_NC * _N, _DEGW), jnp.float32)),
    mesh=_mesh,
    scratch_types=[
        [pltpu.VMEM((_K,), jnp.int32) for _ in range(2)],
        [pltpu.VMEM((_K, _H), jnp.float32) for _ in range(2)],
        [pltpu.VMEM((_K,), jnp.int32) for _ in range(2)],
        pltpu.VMEM((_K, _DEGW), jnp.float32),
        pltpu.VMEM_SHARED((_N, _DEGW), jnp.float32),
        [pltpu.SemaphoreType.DMA for _ in range(2)],
        [pltpu.SemaphoreType.DMA for _ in range(2)],
    ],
)
def _sc_embed_deg(z_hbm, dst_hbm, table_hbm, zeros_hbm, ones_hbm,
                  x_hbm, deg_hbm, zi, rows, di, ones_v, dacc, esem, ssem):
    cid = lax.axis_index("c")
    sid = lax.axis_index("s")
    wid = sid * _NC + cid
    pltpu.sync_copy(zeros_hbm, rows[0])
    for t in range(_TPS):
        c = sid + t * _NS

        @pl.when(c < _NCHUNK)
        def _():
            pltpu.sync_copy(rows[0], dacc.at[pl.ds(c * _K, _K)])

    pltpu.sync_copy(ones_hbm, ones_v)
    plsc.subcore_barrier()

    def chunk_id(t):
        c = wid + t * _NW
        return jnp.where(c < _NCHUNK, c, 0)

    def load_z(t, b):
        pltpu.sync_copy(z_hbm.at[pl.ds(chunk_id(t) * _K, _K)], zi[b])

    def write_x(t, b):
        pltpu.sync_copy(rows[b], x_hbm.at[pl.ds(chunk_id(t) * _K, _K)])

    load_z(0, 0)
    g0 = pltpu.async_copy(table_hbm.at[zi[0]], rows[0], esem[0])
    load_z(1, 1)
    g1 = pltpu.async_copy(table_hbm.at[zi[1]], rows[1], esem[1])
    g0.wait()
    write_x(0, 0)
    load_z(2, 0)
    g2 = pltpu.async_copy(table_hbm.at[zi[0]], rows[0], esem[0])
    g1.wait()
    write_x(1, 1)
    load_z(3, 1)
    g3 = pltpu.async_copy(table_hbm.at[zi[1]], rows[1], esem[1])
    g2.wait()
    write_x(2, 0)
    g3.wait()
    write_x(3, 1)

    ebase = cid * (_E // _NC) + sid * _EPW
    npair = (_EPW // _K) // 2
    nrem = _EPW // _K - 2 * npair

    def dpair(u, carry):
        off = ebase + 2 * u * _K
        pltpu.sync_copy(dst_hbm.at[pl.ds(off, _K)], di[0])
        s0 = pltpu.async_copy(ones_v, dacc.at[di[0]], ssem[0], add=True)
        pltpu.sync_copy(dst_hbm.at[pl.ds(off + _K, _K)], di[1])
        s1 = pltpu.async_copy(ones_v, dacc.at[di[1]], ssem[1], add=True)
        s0.wait()
        s1.wait()
        return carry

    lax.fori_loop(0, npair, dpair, 0)
    for r in range(nrem):
        off = ebase + (2 * npair + r) * _K
        pltpu.sync_copy(dst_hbm.at[pl.ds(off, _K)], di[0])
        pltpu.sync_copy(ones_v, dacc.at[di[0]], add=True)

    plsc.subcore_barrier()
    for t in range(_TPS):
        c = sid + t * _NS

        @pl.when(c < _NCHUNK)
        def _():
            pltpu.sync_copy(dacc.at[pl.ds(c * _K, _K)],
                            deg_hbm.at[pl.ds(cid * _N + c * _K, _K)])


_KA = 128
_CPW = _EPW // _KA
_TAIL = _EPW - _CPW * _KA
_G = 2
_NGRP = _CPW // _G


@functools.partial(
    pl.kernel,
    out_type=jax.ShapeDtypeStruct((_NC * _N, _H), jnp.float32),
    mesh=_mesh,
    scratch_types=[
        [pltpu.VMEM((_KA,), jnp.int32) for _ in range(_G)],
        [pltpu.VMEM((_KA,), jnp.int32) for _ in range(_G)],
        [pltpu.VMEM((_KA, _H), jnp.float32) for _ in range(_G)],
        pltpu.VMEM((_TAIL,), jnp.int32),
        pltpu.VMEM((_TAIL,), jnp.int32),
        pltpu.VMEM((_TAIL, _H), jnp.float32),
        pltpu.VMEM_SHARED((_N, _H), jnp.float32),
        [pltpu.SemaphoreType.DMA for _ in range(_G)],
        pltpu.SemaphoreType.DMA,
    ],
)
def _sc_aggregate(y_hbm, src_hbm, dst_hbm, zeros_hbm, p_hbm,
                  si, di, rows, si_t, di_t, rows_t, acc, gsem, ssem):
    cid = lax.axis_index("c")
    sid = lax.axis_index("s")
    pltpu.sync_copy(zeros_hbm, rows[0].at[pl.ds(0, _K)])
    for t in range(_TPS):
        c = sid + t * _NS

        @pl.when(c < _NCHUNK)
        def _():
            pltpu.sync_copy(rows[0].at[pl.ds(0, _K)],
                            acc.at[pl.ds(c * _K, _K)])

    plsc.subcore_barrier()
    ebase = cid * (_E // _NC) + sid * _EPW

    def group(gbase, nj):
        gds = []
        for j in range(nj):
            off = gbase + j * _KA
            pltpu.sync_copy(src_hbm.at[pl.ds(off, _KA)], si[j])
            pltpu.sync_copy(dst_hbm.at[pl.ds(off, _KA)], di[j])
            gds.append(pltpu.async_copy(y_hbm.at[si[j]], rows[j], gsem[j]))
        sds = []
        for j in range(nj):
            gds[j].wait()
            sds.append(pltpu.async_copy(rows[j], acc.at[di[j]], ssem,
                                        add=True))
        for j in range(nj):
            sds[j].wait()

    def body(g, carry):
        group(ebase + g * (_G * _KA), _G)
        return carry

    lax.fori_loop(0, _NGRP, body, 0)
    offt = ebase + _CPW * _KA
    pltpu.sync_copy(src_hbm.at[pl.ds(offt, _TAIL)], si_t)
    pltpu.sync_copy(dst_hbm.at[pl.ds(offt, _TAIL)], di_t)
    pltpu.async_copy(y_hbm.at[si_t], rows_t, gsem[0]).wait()
    pltpu.sync_copy(rows_t, acc.at[di_t], add=True)
    plsc.subcore_barrier()
    for t in range(_TPS):
        c = sid + t * _NS

        @pl.when(c < _NCHUNK)
        def _():
            pltpu.sync_copy(acc.at[pl.ds(c * _K, _K)],
                            p_hbm.at[pl.ds(cid * _N + c * _K, _K)])


@functools.partial(
    pl.kernel,
    out_type=(jax.ShapeDtypeStruct((2, 2 * _NG, _H), jnp.float32),
              jax.ShapeDtypeStruct((2 * _NG, _H), jnp.float32),
              jax.ShapeDtypeStruct((2, 2 * _NG, _DEGW), jnp.float32)),
    mesh=_mesh,
    scratch_types=[
        pltpu.VMEM((16,), jnp.int32),
        [pltpu.VMEM((16, _H), jnp.float32) for _ in range(5)],
        [pltpu.SemaphoreType.DMA for _ in range(5)],
    ],
)
def _sc_pool(p_hbm, y_hbm, deg_hbm, ci_hbm, gp_hbm, gy_hbm, gd_hbm,
             idx_v, rows, sem):
    cid = lax.axis_index("c")
    sid = lax.axis_index("s")
    wid = sid * _NC + cid

    @pl.when(wid < (2 * _NG) // 16)
    def _():
        pltpu.sync_copy(ci_hbm.at[pl.ds(wid * 16, 16)], idx_v)
        ds = []
        for h in range(2):
            ds.append(pltpu.async_copy(
                p_hbm.at[pl.ds(h * _N, _N)].at[idx_v], rows[h], sem[h]))
            ds.append(pltpu.async_copy(
                deg_hbm.at[pl.ds(h * _N, _N)].at[idx_v], rows[2 + h],
                sem[2 + h]))
        ds.append(pltpu.async_copy(y_hbm.at[idx_v], rows[4], sem[4]))
        for d in ds:
            d.wait()
        for h in range(2):
            pltpu.sync_copy(rows[h], gp_hbm.at[h, pl.ds(wid * 16, 16)])
            pltpu.sync_copy(rows[2 + h], gd_hbm.at[h, pl.ds(wid * 16, 16)])
        pltpu.sync_copy(rows[4], gy_hbm.at[pl.ds(wid * 16, 16)])


_BM = 1000


def _tc_first_body(deg_ref, x_ref, w_ref, y_ref, dinv_ref):
    deg = deg_ref[0, :, :1] + deg_ref[1, :, :1] + 1.0
    dinv = lax.rsqrt(deg)
    dinv_ref[...] = dinv
    y_ref[...] = dinv * jnp.dot(x_ref[...], w_ref[...],
                                preferred_element_type=jnp.float32)


_tc_first = pl.pallas_call(
    _tc_first_body,
    grid=(_N // _BM,),
    in_specs=[
        pl.BlockSpec((2, _BM, _DEGW), lambda i: (0, i, 0)),
        pl.BlockSpec((_BM, _H), lambda i: (i, 0)),
        pl.BlockSpec((_H, _H), lambda i: (0, 0)),
    ],
    out_specs=[
        pl.BlockSpec((_BM, _H), lambda i: (i, 0)),
        pl.BlockSpec((_BM, 1), lambda i: (i, 0)),
    ],
    out_shape=[
        jax.ShapeDtypeStruct((_N, _H), jnp.float32),
        jax.ShapeDtypeStruct((_N, 1), jnp.float32),
    ],
)


def _tc_mid_body(p_ref, y_ref, dinv_ref, b_ref, w_ref, yo_ref):
    dinv = dinv_ref[...]
    x = jnp.maximum(dinv * (p_ref[0] + p_ref[1] + y_ref[...]) + b_ref[...],
                    0.0)
    yo_ref[...] = dinv * jnp.dot(x, w_ref[...],
                                 preferred_element_type=jnp.float32)


_tc_mid = pl.pallas_call(
    _tc_mid_body,
    grid=(_N // _BM,),
    in_specs=[
        pl.BlockSpec((2, _BM, _H), lambda i: (0, i, 0)),
        pl.BlockSpec((_BM, _H), lambda i: (i, 0)),
        pl.BlockSpec((_BM, 1), lambda i: (i, 0)),
        pl.BlockSpec((1, _H), lambda i: (0, 0)),
        pl.BlockSpec((_H, _H), lambda i: (0, 0)),
    ],
    out_specs=pl.BlockSpec((_BM, _H), lambda i: (i, 0)),
    out_shape=jax.ShapeDtypeStruct((_N, _H), jnp.float32),
)


def _tc_tail_body(gp_ref, gy_ref, gd_ref, b_ref, w1_ref, b1_ref,
                  w2_ref, b2_ref, o_ref):
    deg = gd_ref[0, :, :1] + gd_ref[1, :, :1] + 1.0
    dinv = lax.rsqrt(deg)
    x3 = dinv * (gp_ref[0] + gp_ref[1] + gy_ref[...]) + b_ref[...]
    prod = x3[:_NG] * x3[_NG:]
    h = jnp.maximum(jnp.dot(prod, w1_ref[...],
                            preferred_element_type=jnp.float32) + b1_ref[...],
                    0.0)
    o_ref[...] = (jnp.dot(h, w2_ref[...], preferred_element_type=jnp.float32)
                  + b2_ref[...])


_tc_tail = pl.pallas_call(
    _tc_tail_body,
    out_shape=jax.ShapeDtypeStruct((_NG, 1), jnp.float32),
)


def kernel(z, edge_index, batch, z_table, W0, b0, W1, b1, W2, b2,
           lin1_W, lin1_b, lin2_W, lin2_b):
    z = z.astype(jnp.int32)
    src = edge_index[0].astype(jnp.int32)
    dst = edge_index[1].astype(jnp.int32)
    ci = jnp.searchsorted(batch, jnp.arange(_NG, dtype=batch.dtype))
    poolidx = jnp.concatenate([ci, ci + 1]).astype(jnp.int32)
    zeros_h = jnp.zeros((_K, _H), jnp.float32)
    ones16 = jnp.ones((_K, _DEGW), jnp.float32)

    x0, deg2 = _sc_embed_deg(z, dst, z_table, zeros_h, ones16)
    y, dinv = _tc_first(deg2.reshape(_NC, _N, _DEGW), x0, W0)
    p = _sc_aggregate(y, src, dst, zeros_h)
    y = _tc_mid(p.reshape(_NC, _N, _H), y, dinv, b0.reshape(1, _H), W1)
    p = _sc_aggregate(y, src, dst, zeros_h)
    y = _tc_mid(p.reshape(_NC, _N, _H), y, dinv, b1.reshape(1, _H), W2)
    p = _sc_aggregate(y, src, dst, zeros_h)
    gp, gy, gd = _sc_pool(p, y, deg2, poolidx)
    out = _tc_tail(gp, gy, gd, b2.reshape(1, _H), lin1_W,
                   lin1_b.reshape(1, _H), lin2_W, lin2_b.reshape(1, 1))
    return out


# speedup vs baseline: 1.1269x; 1.0288x over previous
import functools

import jax
import jax.numpy as jnp
from jax import lax
from jax.experimental import pallas as pl
from jax.experimental.pallas import tpu as pltpu
from jax.experimental.pallas import tpu_sc as plsc

_N = 10000
_E = 320000
_H = 128
_NG = 200
_NC = 2
_NS = 16
_NW = _NC * _NS
_K = 80
_DEGW = 128
_EPW = _E // _NW
_NCHUNK = _N // _K
_TPS = -(-_NCHUNK // _NS)

_mesh = plsc.VectorSubcoreMesh(core_axis_name="c", subcore_axis_name="s")


@functools.partial(
    pl.kernel,
    out_type=(jax.ShapeDtypeStruct((_N, _H), jnp.float32),
              jax.ShapeDtypeStruct((_NC * _N, _DEGW), jnp.float32)),
    mesh=_mesh,
    scratch_types=[
        [pltpu.VMEM((_K,), jnp.int32) for _ in range(2)],
        [pltpu.VMEM((_K, _H), jnp.float32) for _ in range(2)],
        [pltpu.VMEM((_K,), jnp.int32) for _ in range(2)],
        pltpu.VMEM((_K, _DEGW), jnp.float32),
        pltpu.VMEM_SHARED((_N, _DEGW), jnp.float32),
        [pltpu.SemaphoreType.DMA for _ in range(2)],
        [pltpu.SemaphoreType.DMA for _ in range(2)],
    ],
)
def _sc_embed_deg(z_hbm, dst_hbm, table_hbm, zeros_hbm, ones_hbm,
                  x_hbm, deg_hbm, zi, rows, di, ones_v, dacc, esem, ssem):
    cid = lax.axis_index("c")
    sid = lax.axis_index("s")
    wid = sid * _NC + cid
    pltpu.sync_copy(zeros_hbm, rows[0])
    for t in range(_TPS):
        c = sid + t * _NS

        @pl.when(c < _NCHUNK)
        def _():
            pltpu.sync_copy(rows[0], dacc.at[pl.ds(c * _K, _K)])

    pltpu.sync_copy(ones_hbm, ones_v)
    plsc.subcore_barrier()

    def chunk_id(t):
        c = wid + t * _NW
        return jnp.where(c < _NCHUNK, c, 0)

    def load_z(t, b):
        pltpu.sync_copy(z_hbm.at[pl.ds(chunk_id(t) * _K, _K)], zi[b])

    def write_x(t, b):
        pltpu.sync_copy(rows[b], x_hbm.at[pl.ds(chunk_id(t) * _K, _K)])

    load_z(0, 0)
    g0 = pltpu.async_copy(table_hbm.at[zi[0]], rows[0], esem[0])
    load_z(1, 1)
    g1 = pltpu.async_copy(table_hbm.at[zi[1]], rows[1], esem[1])
    g0.wait()
    write_x(0, 0)
    load_z(2, 0)
    g2 = pltpu.async_copy(table_hbm.at[zi[0]], rows[0], esem[0])
    g1.wait()
    write_x(1, 1)
    load_z(3, 1)
    g3 = pltpu.async_copy(table_hbm.at[zi[1]], rows[1], esem[1])
    g2.wait()
    write_x(2, 0)
    g3.wait()
    write_x(3, 1)

    ebase = cid * (_E // _NC) + sid * _EPW
    npair = (_EPW // _K) // 2
    nrem = _EPW // _K - 2 * npair

    def dpair(u, carry):
        off = ebase + 2 * u * _K
        pltpu.sync_copy(dst_hbm.at[pl.ds(off, _K)], di[0])
        s0 = pltpu.async_copy(ones_v, dacc.at[di[0]], ssem[0], add=True)
        pltpu.sync_copy(dst_hbm.at[pl.ds(off + _K, _K)], di[1])
        s1 = pltpu.async_copy(ones_v, dacc.at[di[1]], ssem[1], add=True)
        s0.wait()
        s1.wait()
        return carry

    lax.fori_loop(0, npair, dpair, 0)
    for r in range(nrem):
        off = ebase + (2 * npair + r) * _K
        pltpu.sync_copy(dst_hbm.at[pl.ds(off, _K)], di[0])
        pltpu.sync_copy(ones_v, dacc.at[di[0]], add=True)

    plsc.subcore_barrier()
    for t in range(_TPS):
        c = sid + t * _NS

        @pl.when(c < _NCHUNK)
        def _():
            pltpu.sync_copy(dacc.at[pl.ds(c * _K, _K)],
                            deg_hbm.at[pl.ds(cid * _N + c * _K, _K)])


_KA = 128
_CPW = _EPW // _KA
_TAIL = _EPW - _CPW * _KA
_G = 2
_NGRP = _CPW // _G


@functools.partial(
    pl.kernel,
    out_type=jax.ShapeDtypeStruct((_NC * _N, _H), jnp.float32),
    mesh=_mesh,
    scratch_types=[
        [pltpu.VMEM((_KA,), jnp.int32) for _ in range(_G)],
        [pltpu.VMEM((_KA,), jnp.int32) for _ in range(_G)],
        [pltpu.VMEM((_KA, _H), jnp.float32) for _ in range(_G)],
        pltpu.VMEM((_TAIL,), jnp.int32),
        pltpu.VMEM((_TAIL,), jnp.int32),
        pltpu.VMEM((_TAIL, _H), jnp.float32),
        pltpu.VMEM_SHARED((_N, _H), jnp.float32),
        [pltpu.SemaphoreType.DMA for _ in range(_G)],
        pltpu.SemaphoreType.DMA,
        pltpu.SemaphoreType.DMA,
    ],
)
def _sc_aggregate(y_hbm, src_hbm, dst_hbm, zeros_hbm, p_hbm,
                  si, di, rows, si_t, di_t, rows_t, acc, gsem, ssem, isem):
    cid = lax.axis_index("c")
    sid = lax.axis_index("s")
    pltpu.sync_copy(zeros_hbm, rows[0].at[pl.ds(0, _K)])
    for t in range(_TPS):
        c = sid + t * _NS

        @pl.when(c < _NCHUNK)
        def _():
            pltpu.sync_copy(rows[0].at[pl.ds(0, _K)],
                            acc.at[pl.ds(c * _K, _K)])

    plsc.subcore_barrier()
    ebase = cid * (_E // _NC) + sid * _EPW

    def group(gbase, nj):
        ids = []
        for j in range(nj):
            off = gbase + j * _KA
            ids.append(pltpu.async_copy(src_hbm.at[pl.ds(off, _KA)], si[j],
                                        isem))
            ids.append(pltpu.async_copy(dst_hbm.at[pl.ds(off, _KA)], di[j],
                                        isem))
        for d in ids:
            d.wait()
        gds = []
        for j in range(nj):
            gds.append(pltpu.async_copy(y_hbm.at[si[j]], rows[j], gsem[j]))
        sds = []
        for j in range(nj):
            gds[j].wait()
            sds.append(pltpu.async_copy(rows[j], acc.at[di[j]], ssem,
                                        add=True))
        for j in range(nj):
            sds[j].wait()

    def body(g, carry):
        group(ebase + g * (_G * _KA), _G)
        return carry

    lax.fori_loop(0, _NGRP, body, 0)
    offt = ebase + _CPW * _KA
    pltpu.sync_copy(src_hbm.at[pl.ds(offt, _TAIL)], si_t)
    pltpu.sync_copy(dst_hbm.at[pl.ds(offt, _TAIL)], di_t)
    pltpu.async_copy(y_hbm.at[si_t], rows_t, gsem[0]).wait()
    pltpu.sync_copy(rows_t, acc.at[di_t], add=True)
    plsc.subcore_barrier()
    for t in range(_TPS):
        c = sid + t * _NS

        @pl.when(c < _NCHUNK)
        def _():
            pltpu.sync_copy(acc.at[pl.ds(c * _K, _K)],
                            p_hbm.at[pl.ds(cid * _N + c * _K, _K)])


@functools.partial(
    pl.kernel,
    out_type=(jax.ShapeDtypeStruct((2, 2 * _NG, _H), jnp.float32),
              jax.ShapeDtypeStruct((2 * _NG, _H), jnp.float32),
              jax.ShapeDtypeStruct((2, 2 * _NG, _DEGW), jnp.float32)),
    mesh=_mesh,
    scratch_types=[
        pltpu.VMEM((16,), jnp.int32),
        [pltpu.VMEM((16, _H), jnp.float32) for _ in range(5)],
        [pltpu.SemaphoreType.DMA for _ in range(5)],
    ],
)
def _sc_pool(p_hbm, y_hbm, deg_hbm, ci_hbm, gp_hbm, gy_hbm, gd_hbm,
             idx_v, rows, sem):
    cid = lax.axis_index("c")
    sid = lax.axis_index("s")
    wid = sid * _NC + cid

    @pl.when(wid < (2 * _NG) // 16)
    def _():
        pltpu.sync_copy(ci_hbm.at[pl.ds(wid * 16, 16)], idx_v)
        ds = []
        for h in range(2):
            ds.append(pltpu.async_copy(
                p_hbm.at[pl.ds(h * _N, _N)].at[idx_v], rows[h], sem[h]))
            ds.append(pltpu.async_copy(
                deg_hbm.at[pl.ds(h * _N, _N)].at[idx_v], rows[2 + h],
                sem[2 + h]))
        ds.append(pltpu.async_copy(y_hbm.at[idx_v], rows[4], sem[4]))
        for d in ds:
            d.wait()
        for h in range(2):
            pltpu.sync_copy(rows[h], gp_hbm.at[h, pl.ds(wid * 16, 16)])
            pltpu.sync_copy(rows[2 + h], gd_hbm.at[h, pl.ds(wid * 16, 16)])
        pltpu.sync_copy(rows[4], gy_hbm.at[pl.ds(wid * 16, 16)])


_BM = 1000


def _tc_first_body(deg_ref, x_ref, w_ref, y_ref, dinv_ref):
    deg = deg_ref[0, :, :1] + deg_ref[1, :, :1] + 1.0
    dinv = lax.rsqrt(deg)
    dinv_ref[...] = dinv
    y_ref[...] = dinv * jnp.dot(x_ref[...], w_ref[...],
                                preferred_element_type=jnp.float32)


_tc_first = pl.pallas_call(
    _tc_first_body,
    grid=(_N // _BM,),
    in_specs=[
        pl.BlockSpec((2, _BM, _DEGW), lambda i: (0, i, 0)),
        pl.BlockSpec((_BM, _H), lambda i: (i, 0)),
        pl.BlockSpec((_H, _H), lambda i: (0, 0)),
    ],
    out_specs=[
        pl.BlockSpec((_BM, _H), lambda i: (i, 0)),
        pl.BlockSpec((_BM, 1), lambda i: (i, 0)),
    ],
    out_shape=[
        jax.ShapeDtypeStruct((_N, _H), jnp.float32),
        jax.ShapeDtypeStruct((_N, 1), jnp.float32),
    ],
)


def _tc_mid_body(p_ref, y_ref, dinv_ref, b_ref, w_ref, yo_ref):
    dinv = dinv_ref[...]
    x = jnp.maximum(dinv * (p_ref[0] + p_ref[1] + y_ref[...]) + b_ref[...],
                    0.0)
    yo_ref[...] = dinv * jnp.dot(x, w_ref[...],
                                 preferred_element_type=jnp.float32)


_tc_mid = pl.pallas_call(
    _tc_mid_body,
    grid=(_N // _BM,),
    in_specs=[
        pl.BlockSpec((2, _BM, _H), lambda i: (0, i, 0)),
        pl.BlockSpec((_BM, _H), lambda i: (i, 0)),
        pl.BlockSpec((_BM, 1), lambda i: (i, 0)),
        pl.BlockSpec((1, _H), lambda i: (0, 0)),
        pl.BlockSpec((_H, _H), lambda i: (0, 0)),
    ],
    out_specs=pl.BlockSpec((_BM, _H), lambda i: (i, 0)),
    out_shape=jax.ShapeDtypeStruct((_N, _H), jnp.float32),
)


def _tc_tail_body(gp_ref, gy_ref, gd_ref, b_ref, w1_ref, b1_ref,
                  w2_ref, b2_ref, o_ref):
    deg = gd_ref[0, :, :1] + gd_ref[1, :, :1] + 1.0
    dinv = lax.rsqrt(deg)
    x3 = dinv * (gp_ref[0] + gp_ref[1] + gy_ref[...]) + b_ref[...]
    prod = x3[:_NG] * x3[_NG:]
    h = jnp.maximum(jnp.dot(prod, w1_ref[...],
                            preferred_element_type=jnp.float32) + b1_ref[...],
                    0.0)
    o_ref[...] = (jnp.dot(h, w2_ref[...], preferred_element_type=jnp.float32)
                  + b2_ref[...])


_tc_tail = pl.pallas_call(
    _tc_tail_body,
    out_shape=jax.ShapeDtypeStruct((_NG, 1), jnp.float32),
)


def kernel(z, edge_index, batch, z_table, W0, b0, W1, b1, W2, b2,
           lin1_W, lin1_b, lin2_W, lin2_b):
    z = z.astype(jnp.int32)
    src = edge_index[0].astype(jnp.int32)
    dst = edge_index[1].astype(jnp.int32)
    ci = jnp.searchsorted(batch, jnp.arange(_NG, dtype=batch.dtype))
    poolidx = jnp.concatenate([ci, ci + 1]).astype(jnp.int32)
    zeros_h = jnp.zeros((_K, _H), jnp.float32)
    ones16 = jnp.ones((_K, _DEGW), jnp.float32)

    x0, deg2 = _sc_embed_deg(z, dst, z_table, zeros_h, ones16)
    y, dinv = _tc_first(deg2.reshape(_NC, _N, _DEGW), x0, W0)
    p = _sc_aggregate(y, src, dst, zeros_h)
    y = _tc_mid(p.reshape(_NC, _N, _H), y, dinv, b0.reshape(1, _H), W1)
    p = _sc_aggregate(y, src, dst, zeros_h)
    y = _tc_mid(p.reshape(_NC, _N, _H), y, dinv, b1.reshape(1, _H), W2)
    p = _sc_aggregate(y, src, dst, zeros_h)
    gp, gy, gd = _sc_pool(p, y, deg2, poolidx)
    out = _tc_tail(gp, gy, gd, b2.reshape(1, _H), lin1_W,
                   lin1_b.reshape(1, _H), lin2_W, lin2_b.reshape(1, 1))
    return out


# speedup vs baseline: 1.1297x; 1.0025x over previous
import functools

import jax
import jax.numpy as jnp
from jax import lax
from jax.experimental import pallas as pl
from jax.experimental.pallas import tpu as pltpu
from jax.experimental.pallas import tpu_sc as plsc

_N = 10000
_E = 320000
_H = 128
_NG = 200
_NC = 2
_NS = 16
_NW = _NC * _NS
_K = 80
_DEGW = 128
_EPW = _E // _NW
_NCHUNK = _N // _K
_TPS = -(-_NCHUNK // _NS)

_mesh = plsc.VectorSubcoreMesh(core_axis_name="c", subcore_axis_name="s")


@functools.partial(
    pl.kernel,
    out_type=(jax.ShapeDtypeStruct((_N, _H), jnp.float32),
              jax.ShapeDtypeStruct((_NC * _N, _DEGW), jnp.float32)),
    mesh=_mesh,
    scratch_types=[
        [pltpu.VMEM((_K,), jnp.int32) for _ in range(2)],
        [pltpu.VMEM((_K, _H), jnp.float32) for _ in range(2)],
        [pltpu.VMEM((_K,), jnp.int32) for _ in range(2)],
        pltpu.VMEM((_K, _DEGW), jnp.float32),
        pltpu.VMEM_SHARED((_N, _DEGW), jnp.float32),
        [pltpu.SemaphoreType.DMA for _ in range(2)],
        [pltpu.SemaphoreType.DMA for _ in range(2)],
        pltpu.SemaphoreType.DMA,
    ],
)
def _sc_embed_deg(z_hbm, dst_hbm, table_hbm, zeros_hbm, ones_hbm,
                  x_hbm, deg_hbm, zi, rows, di, ones_v, dacc, esem, ssem,
                  isem):
    cid = lax.axis_index("c")
    sid = lax.axis_index("s")
    wid = sid * _NC + cid
    pltpu.sync_copy(zeros_hbm, rows[0])
    for t in range(_TPS):
        c = sid + t * _NS

        @pl.when(c < _NCHUNK)
        def _():
            pltpu.sync_copy(rows[0], dacc.at[pl.ds(c * _K, _K)])

    pltpu.sync_copy(ones_hbm, ones_v)
    plsc.subcore_barrier()

    def chunk_id(t):
        c = wid + t * _NW
        return jnp.where(c < _NCHUNK, c, 0)

    def load_z(t, b):
        pltpu.sync_copy(z_hbm.at[pl.ds(chunk_id(t) * _K, _K)], zi[b])

    def write_x(t, b):
        pltpu.sync_copy(rows[b], x_hbm.at[pl.ds(chunk_id(t) * _K, _K)])

    load_z(0, 0)
    g0 = pltpu.async_copy(table_hbm.at[zi[0]], rows[0], esem[0])
    load_z(1, 1)
    g1 = pltpu.async_copy(table_hbm.at[zi[1]], rows[1], esem[1])
    g0.wait()
    write_x(0, 0)
    load_z(2, 0)
    g2 = pltpu.async_copy(table_hbm.at[zi[0]], rows[0], esem[0])
    g1.wait()
    write_x(1, 1)
    load_z(3, 1)
    g3 = pltpu.async_copy(table_hbm.at[zi[1]], rows[1], esem[1])
    g2.wait()
    write_x(2, 0)
    g3.wait()
    write_x(3, 1)

    ebase = cid * (_E // _NC) + sid * _EPW
    npair = (_EPW // _K) // 2
    nrem = _EPW // _K - 2 * npair

    def dpair(u, carry):
        off = ebase + 2 * u * _K
        i0 = pltpu.async_copy(dst_hbm.at[pl.ds(off, _K)], di[0], isem)
        i1 = pltpu.async_copy(dst_hbm.at[pl.ds(off + _K, _K)], di[1], isem)
        i0.wait()
        s0 = pltpu.async_copy(ones_v, dacc.at[di[0]], ssem[0], add=True)
        i1.wait()
        s1 = pltpu.async_copy(ones_v, dacc.at[di[1]], ssem[1], add=True)
        s0.wait()
        s1.wait()
        return carry

    lax.fori_loop(0, npair, dpair, 0)
    for r in range(nrem):
        off = ebase + (2 * npair + r) * _K
        pltpu.sync_copy(dst_hbm.at[pl.ds(off, _K)], di[0])
        pltpu.sync_copy(ones_v, dacc.at[di[0]], add=True)

    plsc.subcore_barrier()
    for t in range(_TPS):
        c = sid + t * _NS

        @pl.when(c < _NCHUNK)
        def _():
            pltpu.sync_copy(dacc.at[pl.ds(c * _K, _K)],
                            deg_hbm.at[pl.ds(cid * _N + c * _K, _K)])


_KA = 128
_CPW = _EPW // _KA
_TAIL = _EPW - _CPW * _KA
_G = 2
_NGRP = _CPW // _G


@functools.partial(
    pl.kernel,
    out_type=jax.ShapeDtypeStruct((_NC * _N, _H), jnp.float32),
    mesh=_mesh,
    scratch_types=[
        [pltpu.VMEM((_KA,), jnp.int32) for _ in range(_G)],
        [pltpu.VMEM((_KA,), jnp.int32) for _ in range(_G)],
        [pltpu.VMEM((_KA, _H), jnp.float32) for _ in range(_G)],
        pltpu.VMEM((_TAIL,), jnp.int32),
        pltpu.VMEM((_TAIL,), jnp.int32),
        pltpu.VMEM((_TAIL, _H), jnp.float32),
        pltpu.VMEM_SHARED((_N, _H), jnp.float32),
        [pltpu.SemaphoreType.DMA for _ in range(_G)],
        pltpu.SemaphoreType.DMA,
        pltpu.SemaphoreType.DMA,
    ],
)
def _sc_aggregate(y_hbm, src_hbm, dst_hbm, zeros_hbm, p_hbm,
                  si, di, rows, si_t, di_t, rows_t, acc, gsem, ssem, isem):
    cid = lax.axis_index("c")
    sid = lax.axis_index("s")
    pltpu.sync_copy(zeros_hbm, rows[0].at[pl.ds(0, _K)])
    for t in range(_TPS):
        c = sid + t * _NS

        @pl.when(c < _NCHUNK)
        def _():
            pltpu.sync_copy(rows[0].at[pl.ds(0, _K)],
                            acc.at[pl.ds(c * _K, _K)])

    plsc.subcore_barrier()
    ebase = cid * (_E // _NC) + sid * _EPW

    def group(gbase, nj):
        ids = []
        for j in range(nj):
            off = gbase + j * _KA
            ids.append(pltpu.async_copy(src_hbm.at[pl.ds(off, _KA)], si[j],
                                        isem))
            ids.append(pltpu.async_copy(dst_hbm.at[pl.ds(off, _KA)], di[j],
                                        isem))
        for d in ids:
            d.wait()
        gds = []
        for j in range(nj):
            gds.append(pltpu.async_copy(y_hbm.at[si[j]], rows[j], gsem[j]))
        sds = []
        for j in range(nj):
            gds[j].wait()
            sds.append(pltpu.async_copy(rows[j], acc.at[di[j]], ssem,
                                        add=True))
        for j in range(nj):
            sds[j].wait()

    def body(g, carry):
        group(ebase + g * (_G * _KA), _G)
        return carry

    lax.fori_loop(0, _NGRP, body, 0)
    offt = ebase + _CPW * _KA
    pltpu.sync_copy(src_hbm.at[pl.ds(offt, _TAIL)], si_t)
    pltpu.sync_copy(dst_hbm.at[pl.ds(offt, _TAIL)], di_t)
    pltpu.async_copy(y_hbm.at[si_t], rows_t, gsem[0]).wait()
    pltpu.sync_copy(rows_t, acc.at[di_t], add=True)
    plsc.subcore_barrier()
    for t in range(_TPS):
        c = sid + t * _NS

        @pl.when(c < _NCHUNK)
        def _():
            pltpu.sync_copy(acc.at[pl.ds(c * _K, _K)],
                            p_hbm.at[pl.ds(cid * _N + c * _K, _K)])


@functools.partial(
    pl.kernel,
    out_type=(jax.ShapeDtypeStruct((2, 2 * _NG, _H), jnp.float32),
              jax.ShapeDtypeStruct((2 * _NG, _H), jnp.float32),
              jax.ShapeDtypeStruct((2, 2 * _NG, _DEGW), jnp.float32)),
    mesh=_mesh,
    scratch_types=[
        pltpu.VMEM((16,), jnp.int32),
        [pltpu.VMEM((16, _H), jnp.float32) for _ in range(5)],
        [pltpu.SemaphoreType.DMA for _ in range(5)],
    ],
)
def _sc_pool(p_hbm, y_hbm, deg_hbm, ci_hbm, gp_hbm, gy_hbm, gd_hbm,
             idx_v, rows, sem):
    cid = lax.axis_index("c")
    sid = lax.axis_index("s")
    wid = sid * _NC + cid

    @pl.when(wid < (2 * _NG) // 16)
    def _():
        pltpu.sync_copy(ci_hbm.at[pl.ds(wid * 16, 16)], idx_v)
        ds = []
        for h in range(2):
            ds.append(pltpu.async_copy(
                p_hbm.at[pl.ds(h * _N, _N)].at[idx_v], rows[h], sem[h]))
            ds.append(pltpu.async_copy(
                deg_hbm.at[pl.ds(h * _N, _N)].at[idx_v], rows[2 + h],
                sem[2 + h]))
        ds.append(pltpu.async_copy(y_hbm.at[idx_v], rows[4], sem[4]))
        for d in ds:
            d.wait()
        for h in range(2):
            pltpu.sync_copy(rows[h], gp_hbm.at[h, pl.ds(wid * 16, 16)])
            pltpu.sync_copy(rows[2 + h], gd_hbm.at[h, pl.ds(wid * 16, 16)])
        pltpu.sync_copy(rows[4], gy_hbm.at[pl.ds(wid * 16, 16)])


_BM = 1000


def _tc_first_body(deg_ref, x_ref, w_ref, y_ref, dinv_ref):
    deg = deg_ref[0, :, :1] + deg_ref[1, :, :1] + 1.0
    dinv = lax.rsqrt(deg)
    dinv_ref[...] = dinv
    y_ref[...] = dinv * jnp.dot(x_ref[...], w_ref[...],
                                preferred_element_type=jnp.float32)


_tc_first = pl.pallas_call(
    _tc_first_body,
    grid=(_N // _BM,),
    in_specs=[
        pl.BlockSpec((2, _BM, _DEGW), lambda i: (0, i, 0)),
        pl.BlockSpec((_BM, _H), lambda i: (i, 0)),
        pl.BlockSpec((_H, _H), lambda i: (0, 0)),
    ],
    out_specs=[
        pl.BlockSpec((_BM, _H), lambda i: (i, 0)),
        pl.BlockSpec((_BM, 1), lambda i: (i, 0)),
    ],
    out_shape=[
        jax.ShapeDtypeStruct((_N, _H), jnp.float32),
        jax.ShapeDtypeStruct((_N, 1), jnp.float32),
    ],
)


def _tc_mid_body(p_ref, y_ref, dinv_ref, b_ref, w_ref, yo_ref):
    dinv = dinv_ref[...]
    x = jnp.maximum(dinv * (p_ref[0] + p_ref[1] + y_ref[...]) + b_ref[...],
                    0.0)
    yo_ref[...] = dinv * jnp.dot(x, w_ref[...],
                                 preferred_element_type=jnp.float32)


_tc_mid = pl.pallas_call(
    _tc_mid_body,
    grid=(_N // _BM,),
    in_specs=[
        pl.BlockSpec((2, _BM, _H), lambda i: (0, i, 0)),
        pl.BlockSpec((_BM, _H), lambda i: (i, 0)),
        pl.BlockSpec((_BM, 1), lambda i: (i, 0)),
        pl.BlockSpec((1, _H), lambda i: (0, 0)),
        pl.BlockSpec((_H, _H), lambda i: (0, 0)),
    ],
    out_specs=pl.BlockSpec((_BM, _H), lambda i: (i, 0)),
    out_shape=jax.ShapeDtypeStruct((_N, _H), jnp.float32),
)


def _tc_tail_body(gp_ref, gy_ref, gd_ref, b_ref, w1_ref, b1_ref,
                  w2_ref, b2_ref, o_ref):
    deg = gd_ref[0, :, :1] + gd_ref[1, :, :1] + 1.0
    dinv = lax.rsqrt(deg)
    x3 = dinv * (gp_ref[0] + gp_ref[1] + gy_ref[...]) + b_ref[...]
    prod = x3[:_NG] * x3[_NG:]
    h = jnp.maximum(jnp.dot(prod, w1_ref[...],
                            preferred_element_type=jnp.float32) + b1_ref[...],
                    0.0)
    o_ref[...] = (jnp.dot(h, w2_ref[...], preferred_element_type=jnp.float32)
                  + b2_ref[...])


_tc_tail = pl.pallas_call(
    _tc_tail_body,
    out_shape=jax.ShapeDtypeStruct((_NG, 1), jnp.float32),
)


def kernel(z, edge_index, batch, z_table, W0, b0, W1, b1, W2, b2,
           lin1_W, lin1_b, lin2_W, lin2_b):
    z = z.astype(jnp.int32)
    src = edge_index[0].astype(jnp.int32)
    dst = edge_index[1].astype(jnp.int32)
    ci = jnp.searchsorted(batch, jnp.arange(_NG, dtype=batch.dtype))
    poolidx = jnp.concatenate([ci, ci + 1]).astype(jnp.int32)
    zeros_h = jnp.zeros((_K, _H), jnp.float32)
    ones16 = jnp.ones((_K, _DEGW), jnp.float32)

    x0, deg2 = _sc_embed_deg(z, dst, z_table, zeros_h, ones16)
    y, dinv = _tc_first(deg2.reshape(_NC, _N, _DEGW), x0, W0)
    p = _sc_aggregate(y, src, dst, zeros_h)
    y = _tc_mid(p.reshape(_NC, _N, _H), y, dinv, b0.reshape(1, _H), W1)
    p = _sc_aggregate(y, src, dst, zeros_h)
    y = _tc_mid(p.reshape(_NC, _N, _H), y, dinv, b1.reshape(1, _H), W2)
    p = _sc_aggregate(y, src, dst, zeros_h)
    gp, gy, gd = _sc_pool(p, y, deg2, poolidx)
    out = _tc_tail(gp, gy, gd, b2.reshape(1, _H), lin1_W,
                   lin1_b.reshape(1, _H), lin2_W, lin2_b.reshape(1, 1))
    return out
